# baseline (speedup 1.0000x reference)
import functools

import jax
import jax.numpy as jnp
from jax import lax
from jax.experimental import pallas as pl
from jax.experimental.pallas import tpu as pltpu
from jax.experimental.pallas import tpu_sc as plsc

BATCH = 16384
HIDDEN = 256
EMBED = 128

_INFO = plsc.get_sparse_core_info()
_NC = _INFO.num_cores
_NS = _INFO.num_subcores
_NW = _NC * _NS
_BPW = BATCH // _NW
_CH = 64
_NCH = _BPW // _CH
_NBUF = 6


def _make_sc_gather():
  mesh = plsc.VectorSubcoreMesh(core_axis_name="c", subcore_axis_name="s")

  @functools.partial(
      pl.kernel,
      mesh=mesh,
      out_type=jax.ShapeDtypeStruct((BATCH, HIDDEN), jnp.float32),
      scratch_types=[pltpu.VMEM((_NCH, _CH), jnp.int32)]
      + [pltpu.VMEM((_CH, HIDDEN), jnp.float32) for _ in range(_NBUF)]
      + [pltpu.SemaphoreType.DMA, pltpu.SemaphoreType.DMA],
  )
  def gather(table_hbm, idx_hbm, out_hbm, idx_v, *rest):
    bufs = rest[:_NBUF]
    sem_in, sem_out = rest[_NBUF], rest[_NBUF + 1]
    wid = lax.axis_index("s") * _NC + lax.axis_index("c")
    base = wid * _BPW
    pltpu.sync_copy(idx_hbm.at[wid], idx_v)
    in_flight = [None] * _NCH
    out_flight = [None] * _NCH
    for c in range(min(_NBUF, _NCH)):
      in_flight[c] = pltpu.async_copy(table_hbm.at[idx_v.at[c]],
                                      bufs[c % _NBUF], sem_in)
    for c in range(_NCH):
      in_flight[c].wait()
      out_flight[c] = pltpu.async_copy(
          bufs[c % _NBUF], out_hbm.at[pl.ds(base + c * _CH, _CH)], sem_out)
      nc = c + _NBUF
      if nc < _NCH:
        out_flight[nc - _NBUF].wait()
        in_flight[nc] = pltpu.async_copy(table_hbm.at[idx_v.at[nc]],
                                         bufs[nc % _NBUF], sem_in)
    for c in range(max(0, _NCH - _NBUF), _NCH):
      out_flight[c].wait()

  return gather


_sc_gather = _make_sc_gather()


def _proj_body(x_ref, w_ref, b_ref, o_ref):
  o_ref[...] = (
      lax.dot_general(x_ref[...], w_ref[...], (((1,), (1,)), ((), ())),
                      preferred_element_type=jnp.float32)
      + b_ref[...]
  )


def _tc_proj(rows, W, b2):
  blk = 8192
  grid = BATCH // blk
  return pl.pallas_call(
      _proj_body,
      grid=(grid,),
      in_specs=[
          pl.BlockSpec((blk, HIDDEN), lambda i: (i, 0)),
          pl.BlockSpec((EMBED, HIDDEN), lambda i: (0, 0)),
          pl.BlockSpec((1, EMBED), lambda i: (0, 0)),
      ],
      out_specs=pl.BlockSpec((blk, EMBED), lambda i: (i, 0)),
      out_shape=jax.ShapeDtypeStruct((BATCH, EMBED), jnp.float32),
  )(rows, W, b2)


def kernel(t, pos_enc, W, b):
  idx = t.reshape(_NW, _NCH, _CH)
  rows = _sc_gather(pos_enc, idx)
  return _tc_proj(rows, W, b.reshape(1, EMBED))


# speedup vs baseline: 1.0053x; 1.0053x over previous
import functools

import jax
import jax.numpy as jnp
from jax import lax
from jax.experimental import pallas as pl
from jax.experimental.pallas import tpu as pltpu
from jax.experimental.pallas import tpu_sc as plsc

BATCH = 16384
HIDDEN = 256
EMBED = 128

_INFO = plsc.get_sparse_core_info()
_NC = _INFO.num_cores
_NS = _INFO.num_subcores
_NW = _NC * _NS
_BPW = BATCH // _NW
_CH = 64
_NCH = _BPW // _CH
_NBUF = 6


def _make_sc_gather():
  mesh = plsc.VectorSubcoreMesh(core_axis_name="c", subcore_axis_name="s")

  @functools.partial(
      pl.kernel,
      mesh=mesh,
      out_type=jax.ShapeDtypeStruct((BATCH, HIDDEN), jnp.float32),
      scratch_types=[pltpu.VMEM((_NCH, _CH), jnp.int32)]
      + [pltpu.VMEM((_CH, HIDDEN), jnp.float32) for _ in range(_NBUF)]
      + [pltpu.SemaphoreType.DMA, pltpu.SemaphoreType.DMA],
  )
  def gather(table_hbm, idx_hbm, out_hbm, idx_v, *rest):
    bufs = rest[:_NBUF]
    sem_in, sem_out = rest[_NBUF], rest[_NBUF + 1]
    wid = lax.axis_index("s") * _NC + lax.axis_index("c")
    base = wid * _BPW
    pltpu.sync_copy(idx_hbm.at[wid], idx_v)
    in_flight = [None] * _NCH
    out_flight = [None] * _NCH
    for c in range(min(_NBUF, _NCH)):
      in_flight[c] = pltpu.async_copy(table_hbm.at[idx_v.at[c]],
                                      bufs[c % _NBUF], sem_in)
    for c in range(_NCH):
      in_flight[c].wait()
      out_flight[c] = pltpu.async_copy(
          bufs[c % _NBUF], out_hbm.at[pl.ds(base + c * _CH, _CH)], sem_out)
      nc = c + _NBUF
      if nc < _NCH:
        out_flight[nc - _NBUF].wait()
        in_flight[nc] = pltpu.async_copy(table_hbm.at[idx_v.at[nc]],
                                         bufs[nc % _NBUF], sem_in)
    for c in range(max(0, _NCH - _NBUF), _NCH):
      out_flight[c].wait()

  return gather


_sc_gather = _make_sc_gather()


def _proj_body(x_ref, w_ref, b_ref, o_ref):
  o_ref[...] = (
      lax.dot_general(x_ref[...], w_ref[...], (((1,), (1,)), ((), ())),
                      preferred_element_type=jnp.float32)
      + b_ref[...]
  )


def _tc_proj(rows, W, b2):
  blk = 8192
  grid = BATCH // blk
  return pl.pallas_call(
      _proj_body,
      grid=(grid,),
      in_specs=[
          pl.BlockSpec((blk, HIDDEN), lambda i: (i, 0)),
          pl.BlockSpec((EMBED, HIDDEN), lambda i: (0, 0)),
          pl.BlockSpec((1, EMBED), lambda i: (0, 0)),
      ],
      out_specs=pl.BlockSpec((blk, EMBED), lambda i: (i, 0)),
      out_shape=jax.ShapeDtypeStruct((BATCH, EMBED), jnp.float32),
      compiler_params=pltpu.CompilerParams(
          dimension_semantics=("parallel",)),
  )(rows, W, b2)


def kernel(t, pos_enc, W, b):
  idx = t.reshape(_NW, _NCH, _CH)
  rows = _sc_gather(pos_enc, idx)
  return _tc_proj(rows, W, b.reshape(1, EMBED))
